# baseline (speedup 1.0000x reference)
import functools

import jax
import jax.numpy as jnp
from jax import lax
from jax.experimental import pallas as pl
from jax.experimental.pallas import tpu as pltpu
from jax.experimental.pallas import tpu_sc as plsc

VOCAB = 100000
EMB = 1024
SEQ = 2048
BATCH = 4

NUM_CORES = 2
NUM_SUBCORES = 16
NUM_WORKERS = NUM_CORES * NUM_SUBCORES
ROWS_TOTAL = BATCH * SEQ
ROWS_PER_W = ROWS_TOTAL // NUM_WORKERS
CHUNK = 8
NCHUNK = ROWS_PER_W // CHUNK
VEC_PER_ROW = EMB // 16
VEC_PER_CHUNK = CHUNK * VEC_PER_ROW
NGB = 3
NPB = 4
SH_ROWS = 4 * ROWS_PER_W

_mesh = plsc.VectorSubcoreMesh(
    core_axis_name="c", subcore_axis_name="s",
    num_cores=NUM_CORES, num_subcores=NUM_SUBCORES,
)


def _add_chunk(pb, gb):
    def body(i, carry):
        r = i // VEC_PER_ROW
        c = (i - r * VEC_PER_ROW) * 16
        plsc.addupdate(pb.at[r, pl.ds(c, 16)], gb[r, pl.ds(c, 16)])
        return carry
    lax.fori_loop(0, VEC_PER_CHUNK, body, 0, unroll=8)


@functools.partial(
    pl.kernel,
    out_type=jax.ShapeDtypeStruct((ROWS_TOTAL, EMB), jnp.float32),
    mesh=_mesh,
    scratch_types=[
        pltpu.VMEM((NCHUNK, CHUNK), jnp.int32),
        pltpu.VMEM_SHARED((SH_ROWS, EMB), jnp.float32),
        pltpu.VMEM((CHUNK, EMB), jnp.float32),
        pltpu.VMEM((CHUNK, EMB), jnp.float32),
        pltpu.VMEM((CHUNK, EMB), jnp.float32),
        pltpu.VMEM((CHUNK, EMB), jnp.float32),
        pltpu.VMEM((CHUNK, EMB), jnp.float32),
        pltpu.VMEM((CHUNK, EMB), jnp.float32),
        pltpu.VMEM((CHUNK, EMB), jnp.float32),
        [pltpu.SemaphoreType.DMA] * (NGB + 2 * NPB),
    ],
)
def _embed_sc(ids_hbm, table_hbm, pos_hbm, out_hbm,
              idx_v, shpos,
              gb0, gb1, gb2, pb0, pb1, pb2, pb3, sems):
    cid = lax.axis_index("c")
    sid = lax.axis_index("s")
    wid = sid * NUM_CORES + cid
    base = wid * ROWS_PER_W
    sh_base = (sid % 4) * ROWS_PER_W

    pltpu.sync_copy(ids_hbm.at[wid], idx_v)

    gbufs = (gb0, gb1, gb2)
    pbufs = (pb0, pb1, pb2, pb3)

    descs = {}
    stores = [None] * NPB

    def prefetch_gather(j):
        slot = j % NGB
        descs[("g", j)] = pltpu.async_copy(
            table_hbm.at[idx_v.at[j]], gbufs[slot], sems[slot])

    def prefetch_pos(j):
        slot = j % NPB
        if stores[slot] is not None:
            stores[slot].wait()
            stores[slot] = None
        descs[("p", j)] = pltpu.async_copy(
            shpos.at[pl.ds(sh_base + j * CHUNK, CHUNK)], pbufs[slot],
            sems[NGB + slot])

    prefetch_gather(0)
    prefetch_gather(1)

    hbm_row0 = (sid % 4) * 512 + cid * ROWS_PER_W + (sid // 4) * 64
    sh_row0 = (sid % 4) * ROWS_PER_W + (sid // 4) * 64
    pltpu.sync_copy(pos_hbm.at[pl.ds(hbm_row0, 64)],
                    shpos.at[pl.ds(sh_row0, 64)])
    plsc.subcore_barrier()

    prefetch_pos(0)
    prefetch_pos(1)
    for j in range(NCHUNK):
        pslot = j % NPB
        if j + 2 < NCHUNK:
            prefetch_gather(j + 2)
            prefetch_pos(j + 2)
        descs.pop(("g", j)).wait()
        descs.pop(("p", j)).wait()
        _add_chunk(pbufs[pslot], gbufs[j % NGB])
        stores[pslot] = pltpu.async_copy(
            pbufs[pslot], out_hbm.at[pl.ds(base + j * CHUNK, CHUNK)],
            sems[NGB + NPB + pslot])
    for st in stores:
        if st is not None:
            st.wait()


def kernel(input_ids, token_table, position_embedding):
    ids = input_ids.astype(jnp.int32).reshape(NUM_WORKERS, NCHUNK, CHUNK)
    pos = position_embedding.reshape(SEQ, EMB)
    out = _embed_sc(ids, token_table, pos)
    return out.reshape(BATCH, SEQ, EMB)


# speedup vs baseline: 1.4594x; 1.4594x over previous
import functools

import jax
import jax.numpy as jnp
from jax import lax
from jax.experimental import pallas as pl
from jax.experimental.pallas import tpu as pltpu
from jax.experimental.pallas import tpu_sc as plsc

VOCAB = 100000
EMB = 1024
SEQ = 2048
BATCH = 4

NUM_CORES = 2
NUM_SUBCORES = 16
NUM_WORKERS = NUM_CORES * NUM_SUBCORES
ROWS_TOTAL = BATCH * SEQ
ROWS_PER_W = ROWS_TOTAL // NUM_WORKERS
CHUNK = 16
NCHUNK = ROWS_PER_W // CHUNK
VEC_PER_ROW = EMB // 16
VEC_PER_CHUNK = CHUNK * VEC_PER_ROW

_mesh = plsc.VectorSubcoreMesh(
    core_axis_name="c", subcore_axis_name="s",
    num_cores=NUM_CORES, num_subcores=NUM_SUBCORES,
)


def _add_chunk(pb, gb):
    def body(i, carry):
        r = i // VEC_PER_ROW
        c = (i - r * VEC_PER_ROW) * 16
        plsc.addupdate(pb.at[r, pl.ds(c, 16)], gb[r, pl.ds(c, 16)])
        return carry
    lax.fori_loop(0, VEC_PER_CHUNK, body, 0, unroll=8)


@functools.partial(
    pl.kernel,
    out_type=jax.ShapeDtypeStruct((ROWS_TOTAL, EMB), jnp.float32),
    mesh=_mesh,
    scratch_types=[
        pltpu.VMEM((NCHUNK, CHUNK), jnp.int32),
        pltpu.VMEM((CHUNK, EMB), jnp.float32),
        pltpu.VMEM((CHUNK, EMB), jnp.float32),
        pltpu.VMEM((CHUNK, EMB), jnp.float32),
        pltpu.VMEM((CHUNK, EMB), jnp.float32),
        pltpu.SemaphoreType.DMA,
        pltpu.SemaphoreType.DMA,
        pltpu.SemaphoreType.DMA,
        pltpu.SemaphoreType.DMA,
        pltpu.SemaphoreType.DMA,
        pltpu.SemaphoreType.DMA,
    ],
)
def _embed_sc(ids_hbm, table_hbm, pos_hbm, out_hbm,
              idx_v, pb0, pb1, gb0, gb1,
              psem0, psem1, gsem0, gsem1, ssem0, ssem1):
    wid = lax.axis_index("s") * NUM_CORES + lax.axis_index("c")
    base = wid * ROWS_PER_W
    pos_base = base % SEQ

    pltpu.sync_copy(ids_hbm.at[wid], idx_v)

    pbufs = (pb0, pb1)
    gbufs = (gb0, gb1)
    psems = (psem0, psem1)
    gsems = (gsem0, gsem1)
    ssems = (ssem0, ssem1)

    descs = {}
    stores = [None, None]

    def prefetch(j):
        slot = j % 2
        gd = pltpu.async_copy(table_hbm.at[idx_v.at[j]], gbufs[slot],
                              gsems[slot])
        pd = pltpu.async_copy(pos_hbm.at[pl.ds(pos_base + j * CHUNK, CHUNK)],
                              pbufs[slot], psems[slot])
        descs[j] = (gd, pd)

    prefetch(0)
    for j in range(NCHUNK):
        slot = j % 2
        nxt = (j + 1) % 2
        if j + 1 < NCHUNK:
            if stores[nxt] is not None:
                stores[nxt].wait()
                stores[nxt] = None
            prefetch(j + 1)
        gd, pd = descs.pop(j)
        gd.wait()
        pd.wait()
        _add_chunk(pbufs[slot], gbufs[slot])
        stores[slot] = pltpu.async_copy(
            pbufs[slot], out_hbm.at[pl.ds(base + j * CHUNK, CHUNK)],
            ssems[slot])
    stores[0].wait()
    stores[1].wait()


def kernel(input_ids, token_table, position_embedding):
    ids = input_ids.astype(jnp.int32).reshape(NUM_WORKERS, NCHUNK, CHUNK)
    pos = position_embedding.reshape(SEQ, EMB)
    out = _embed_sc(ids, token_table, pos)
    return out.reshape(BATCH, SEQ, EMB)


# speedup vs baseline: 1.5126x; 1.0364x over previous
import functools

import jax
import jax.numpy as jnp
from jax import lax
from jax.experimental import pallas as pl
from jax.experimental.pallas import tpu as pltpu
from jax.experimental.pallas import tpu_sc as plsc

VOCAB = 100000
EMB = 1024
SEQ = 2048
BATCH = 4

NUM_CORES = 2
NUM_SUBCORES = 16
NUM_WORKERS = NUM_CORES * NUM_SUBCORES
ROWS_TOTAL = BATCH * SEQ
ROWS_PER_W = ROWS_TOTAL // NUM_WORKERS
CHUNK = 16
NCHUNK = ROWS_PER_W // CHUNK
VEC_PER_ROW = EMB // 16
VEC_PER_CHUNK = CHUNK * VEC_PER_ROW
NGB = 3
NPB = 4

_mesh = plsc.VectorSubcoreMesh(
    core_axis_name="c", subcore_axis_name="s",
    num_cores=NUM_CORES, num_subcores=NUM_SUBCORES,
)


def _add_chunk(pb, gb):
    def body(i, carry):
        r = i // VEC_PER_ROW
        c = (i - r * VEC_PER_ROW) * 16
        plsc.addupdate(pb.at[r, pl.ds(c, 16)], gb[r, pl.ds(c, 16)])
        return carry
    lax.fori_loop(0, VEC_PER_CHUNK, body, 0, unroll=8)


@functools.partial(
    pl.kernel,
    out_type=jax.ShapeDtypeStruct((ROWS_TOTAL, EMB), jnp.float32),
    mesh=_mesh,
    scratch_types=[
        pltpu.VMEM((NCHUNK, CHUNK), jnp.int32),
        [pltpu.VMEM((CHUNK, EMB), jnp.float32)] * NGB,
        [pltpu.VMEM((CHUNK, EMB), jnp.float32)] * NPB,
        [pltpu.SemaphoreType.DMA] * (NGB + 2 * NPB),
    ],
)
def _embed_sc(ids_hbm, table_hbm, pos_hbm, out_hbm,
              idx_v, gbufs, pbufs, sems):
    wid = lax.axis_index("s") * NUM_CORES + lax.axis_index("c")
    base = wid * ROWS_PER_W
    pos_base = base % SEQ

    pltpu.sync_copy(ids_hbm.at[wid], idx_v)

    descs = {}
    stores = [None] * NPB

    def prefetch_gather(j):
        descs[("g", j)] = pltpu.async_copy(
            table_hbm.at[idx_v.at[j]], gbufs[j % NGB], sems[j % NGB])

    def prefetch_pos(j):
        slot = j % NPB
        if stores[slot] is not None:
            stores[slot].wait()
            stores[slot] = None
        descs[("p", j)] = pltpu.async_copy(
            pos_hbm.at[pl.ds(pos_base + j * CHUNK, CHUNK)], pbufs[slot],
            sems[NGB + slot])

    prefetch_gather(0)
    prefetch_gather(1)
    prefetch_pos(0)
    prefetch_pos(1)
    for j in range(NCHUNK):
        pslot = j % NPB
        if j + 2 < NCHUNK:
            prefetch_gather(j + 2)
            prefetch_pos(j + 2)
        descs.pop(("g", j)).wait()
        descs.pop(("p", j)).wait()
        _add_chunk(pbufs[pslot], gbufs[j % NGB])
        stores[pslot] = pltpu.async_copy(
            pbufs[pslot], out_hbm.at[pl.ds(base + j * CHUNK, CHUNK)],
            sems[NGB + NPB + pslot])
    for st in stores:
        if st is not None:
            st.wait()


def kernel(input_ids, token_table, position_embedding):
    ids = input_ids.astype(jnp.int32).reshape(NUM_WORKERS, NCHUNK, CHUNK)
    pos = position_embedding.reshape(SEQ, EMB)
    out = _embed_sc(ids, token_table, pos)
    return out.reshape(BATCH, SEQ, EMB)
